# baseline (speedup 1.0000x reference)
import functools

import jax
import jax.numpy as jnp
from jax import lax
from jax.experimental import pallas as pl
from jax.experimental.pallas import tpu as pltpu
from jax.experimental.pallas import tpu_sc as plsc
from jax._src import config as _jcfg

N = 10000
D = 128
E = 320000
PARAM_H = 1

NT = 32
NP = 10240
EPC = 128
ECH = 80
EPT = ECH * EPC
PPC = 64
PCH = 5
RPT = NP // 16

_f32 = jnp.float32
_i32 = jnp.int32


def _mesh():
    return plsc.VectorSubcoreMesh(core_axis_name="c", subcore_axis_name="s")


def _i(v):
    v = jnp.asarray(v)
    return v if v.dtype == _i32 else v.astype(_i32)


def _zero_rows(buf, nrows):
    def body(i, _):
        i = _i(i)
        for r in range(D // 16):
            buf[i, pl.ds(r * 16, 16)] = jnp.zeros((16,), _f32)
        return 0
    lax.fori_loop(0, nrows, body, 0)


SB = 16
ECH0 = 160
EPT0 = ECH0 * EPC
EPAD = 16 * EPT0


@functools.partial(
    pl.kernel,
    out_type=jax.ShapeDtypeStruct((NP, D), _f32),
    mesh=_mesh(),
    compiler_params=pltpu.CompilerParams(needs_layout_passes=False),
    scratch_types=[
        pltpu.VMEM_SHARED((NP, D), _f32),
        pltpu.VMEM((SB, EPC), _i32),
        pltpu.VMEM((SB, EPC), _i32),
        pltpu.VMEM((SB * EPC,), _f32),
        pltpu.VMEM((EPC, D), _f32),
        pltpu.VMEM((EPC, D), _f32),
        pltpu.SemaphoreType.DMA,
        pltpu.SemaphoreType.DMA,
        pltpu.SemaphoreType.DMA,
        pltpu.SemaphoreType.DMA,
    ],
)
def _propagate(table, src3, dst3, w3, out,
               accum, sidx, didx, wv, rows0, rows1, g0, g1, s0, s1):
    c = lax.axis_index("c")
    s = lax.axis_index("s")

    def scale(buf, cb):
        def edge(e):
            wspl = plsc.load_gather(wv, [jnp.full((16,), cb * EPC + e, _i32)])
            for r in range(D // 16):
                buf[e, pl.ds(r * 16, 16)] = buf[e, pl.ds(r * 16, 16)] * wspl
        plsc.parallel_loop(0, EPC, unroll=8)(edge)

    base = s * RPT

    @pl.when(c == 0)
    def _():
        _zero_rows(rows0, EPC)
        for kk in range(RPT // EPC):
            pltpu.sync_copy(rows0, accum.at[pl.ds(base + kk * EPC, EPC)])

    plsc.subcore_barrier()

    @pl.when(c == 0)
    def _():
        def block(blk, _):
            blk = _i(blk)
            pltpu.sync_copy(src3.at[s, pl.ds(blk * SB, SB)], sidx)
            pltpu.sync_copy(dst3.at[s, pl.ds(blk * SB, SB)], didx)
            pltpu.sync_copy(w3.at[s, pl.ds(blk * (SB * EPC), SB * EPC)], wv)

            bufs = (rows0, rows1)
            gsems = (g0, g1)
            ssems = (s0, s1)
            pltpu.async_copy(table.at[sidx.at[0]], rows0, g0)
            pltpu.async_copy(table.at[sidx.at[1]], rows1, g1)

            def pair(t, _):
                t = _i(t)
                sd = [None, None]
                for b in range(2):
                    cb = 2 * t + b
                    pltpu.make_async_copy(
                        table.at[sidx.at[cb]], bufs[b], gsems[b]).wait()
                    scale(bufs[b], cb)
                    sd[b] = pltpu.async_copy(
                        bufs[b], accum.at[didx.at[cb]], ssems[b], add=True)
                for b in range(2):
                    sd[b].wait()

                    @pl.when(t < SB // 2 - 1)
                    def _():
                        pltpu.async_copy(
                            table.at[sidx.at[2 * t + 2 + b]], bufs[b],
                            gsems[b])
                return 0
            lax.fori_loop(0, SB // 2, pair, 0)
            return 0
        lax.fori_loop(0, ECH0 // SB, block, 0)

    plsc.subcore_barrier()

    @pl.when(c == 0)
    def _():
        pltpu.sync_copy(accum.at[pl.ds(base, RPT)], out.at[pl.ds(base, RPT)])


@functools.partial(
    pl.kernel,
    out_type=jax.ShapeDtypeStruct((NT, EPT), _i32),
    mesh=_mesh(),
    compiler_params=pltpu.CompilerParams(needs_layout_passes=False),
    scratch_types=[
        pltpu.VMEM((NP,), _i32),
        pltpu.VMEM((EPT,), _i32),
        pltpu.VMEM((EPT,), _i32),
    ],
)
def _compose(seg, src3, out, segtab, sidx, res):
    c = lax.axis_index("c")
    s = lax.axis_index("s")
    wid = c * 16 + s
    pltpu.sync_copy(seg, segtab)
    pltpu.sync_copy(src3.at[wid], sidx)

    def body(t):
        sv = sidx[pl.ds(t * 16, 16)]
        res[pl.ds(t * 16, 16)] = plsc.load_gather(segtab, [sv])
    plsc.parallel_loop(0, EPT // 16, unroll=8)(body)
    pltpu.sync_copy(res, out.at[wid])


@functools.partial(
    pl.kernel,
    out_type=(
        jax.ShapeDtypeStruct((2, NP, D), _f32),
        jax.ShapeDtypeStruct((2, NP), _f32),
        jax.ShapeDtypeStruct((NP,), _i32),
    ),
    mesh=_mesh(),
    compiler_params=pltpu.CompilerParams(needs_layout_passes=False),
    scratch_types=[
        pltpu.VMEM_SHARED((NP, D), _f32),
        pltpu.VMEM_SHARED((NP,), _f32),
        pltpu.VMEM((PCH, PPC), _i32),
        pltpu.VMEM((PCH, PPC), _i32),
        pltpu.VMEM((PPC, D), _f32),
        pltpu.VMEM((PPC,), _f32),
        pltpu.VMEM((RPT,), _f32),
    ],
)
def _segsum(p0, ord3, seg3, sums, cnts, segarr,
            accum, cacc, ordv, segv, buf, ones, zbuf):
    c = lax.axis_index("c")
    s = lax.axis_index("s")
    wid = c * 16 + s

    _zero_rows(buf, PPC)
    def zb(i, _):
        i = _i(i)
        zbuf[pl.ds(i * 16, 16)] = jnp.zeros((16,), _f32)
        return 0
    lax.fori_loop(0, RPT // 16, zb, 0)
    for r in range(PPC // 16):
        ones[pl.ds(r * 16, 16)] = jnp.ones((16,), _f32)

    base = s * RPT
    for kk in range(RPT // PPC):
        pltpu.sync_copy(buf, accum.at[pl.ds(base + kk * PPC, PPC)])
    pltpu.sync_copy(zbuf, cacc.at[pl.ds(base, RPT)])

    pltpu.sync_copy(ord3.at[wid], ordv)
    pltpu.sync_copy(seg3.at[wid], segv)
    plsc.subcore_barrier()

    for k in range(PCH):
        pltpu.sync_copy(p0.at[ordv.at[k]], buf)
        pltpu.sync_copy(buf, accum.at[segv.at[k]], add=True)
        pltpu.sync_copy(ones, cacc.at[segv.at[k]], add=True)
        pltpu.sync_copy(segv.at[k], segarr.at[ordv.at[k]])

    plsc.subcore_barrier()
    pltpu.sync_copy(accum.at[pl.ds(base, RPT)], sums.at[c, pl.ds(base, RPT)])
    pltpu.sync_copy(cacc.at[pl.ds(base, RPT)], cnts.at[c, pl.ds(base, RPT)])


@functools.partial(
    pl.kernel,
    out_type=jax.ShapeDtypeStruct((NP, D), _f32),
    mesh=_mesh(),
    compiler_params=pltpu.CompilerParams(needs_layout_passes=False),
    scratch_types=[
        pltpu.VMEM((PCH, PPC), _i32),
        pltpu.VMEM((PPC, D), _f32),
    ],
)
def _rowgather(table, idx3, out, idxv, buf):
    c = lax.axis_index("c")
    s = lax.axis_index("s")
    wid = c * 16 + s
    pltpu.sync_copy(idx3.at[wid], idxv)
    base = wid * (PCH * PPC)
    for k in range(PCH):
        pltpu.sync_copy(table.at[idxv.at[k]], buf)
        pltpu.sync_copy(buf, out.at[pl.ds(base + k * PPC, PPC)])


_MM_BLOCK = 640


def _mm_kernel(do_relu, s0_ref, s1_ref, c0_ref, c1_ref, wt_ref, b_ref, o_ref):
    cnt = jnp.maximum(c0_ref[...] + c1_ref[...], 1.0)
    xs = (s0_ref[...] + s1_ref[...]) / cnt
    acc = jnp.dot(xs, wt_ref[...], preferred_element_type=jnp.float32)
    acc = acc + b_ref[...][None, :]
    if do_relu:
        acc = jnp.maximum(acc, 0.0)
    o_ref[...] = acc


def _c0():
    return jnp.zeros((), jnp.int32)


def _mean_matmul(s0, s1, c0, c1, Wt, b, do_relu):
    n = s0.shape[0]
    grid = n // _MM_BLOCK
    return pl.pallas_call(
        functools.partial(_mm_kernel, do_relu),
        grid=(grid,),
        in_specs=[
            pl.BlockSpec((_MM_BLOCK, D), lambda i: (i, _c0())),
            pl.BlockSpec((_MM_BLOCK, D), lambda i: (i, _c0())),
            pl.BlockSpec((_MM_BLOCK, 1), lambda i: (i, _c0())),
            pl.BlockSpec((_MM_BLOCK, 1), lambda i: (i, _c0())),
            pl.BlockSpec((D, D), lambda i: (_c0(), _c0())),
            pl.BlockSpec((D,), lambda i: (_c0(),)),
        ],
        out_specs=pl.BlockSpec((_MM_BLOCK, D), lambda i: (i, _c0())),
        out_shape=jax.ShapeDtypeStruct((n, D), jnp.float32),
    )(s0, s1, c0[:, None], c1[:, None], Wt, b)


def _sorted_groups(p0, wide):
    h = p0[:N]
    itype = jnp.int64 if wide else jnp.int32
    q = jnp.round(h * PARAM_H).astype(itype)
    wts = jnp.arange(D, dtype=itype) * itype(2654435761) + itype(1)
    code = (q * wts).sum(axis=1)
    if wide:
        lo = code.astype(jnp.int32)
        hi = (code >> 32).astype(jnp.int32)
    else:
        lo = code
        hi = jnp.zeros((N,), jnp.int32)
    iota = jnp.arange(N, dtype=jnp.int32)
    lo_s, hi_s, order = lax.sort((lo, hi, iota), num_keys=1)
    newseg = ((lo_s[1:] != lo_s[:-1]) | (hi_s[1:] != hi_s[:-1]))
    seg_sorted = jnp.concatenate(
        [jnp.zeros((1,), jnp.int32), jnp.cumsum(newseg.astype(jnp.int32))])
    pad = jnp.arange(N, NP, dtype=jnp.int32)
    order_p = jnp.concatenate([order, pad]).reshape(NT, PCH, PPC)
    seg_p = jnp.concatenate([seg_sorted, pad]).reshape(NT, PCH, PPC)
    return order_p, seg_p


def kernel(x, edge_index, edge_weight, vertex_cnt, rule_cnt, W1, b1, W2, b2):
    wide = edge_index.dtype == jnp.int64
    with _jcfg.enable_x64(False):
        src = edge_index[0].astype(jnp.int32)
        dst = edge_index[1].astype(jnp.int32)
        epad = EPAD - E
        srcflat = jnp.concatenate([src, jnp.zeros((epad,), jnp.int32)])
        dstflat = jnp.concatenate([dst, jnp.zeros((epad,), jnp.int32)])
        wflat = jnp.concatenate([edge_weight, jnp.zeros((epad,),
                                                        jnp.float32)])
        srcf = srcflat.reshape(NT, EPT)
        srcA = srcflat.reshape(16, ECH0, EPC)
        dstA = dstflat.reshape(16, ECH0, EPC)
        wA = wflat.reshape(16, EPT0)

        pp = _propagate(x, srcA, dstA, wA)
    ord1, segs1 = _sorted_groups(pp, wide)
    with _jcfg.enable_x64(False):
        sums1, cnts1, seg1 = _segsum(pp, ord1, segs1)
        src2A = _compose(seg1, srcf).reshape(16, ECH0, EPC)
        h1 = _mean_matmul(sums1[0], sums1[1], cnts1[0], cnts1[1],
                          W1.T, b1, do_relu=True)
        pp2 = _propagate(h1, src2A, dstA, wA)
    ord2, segs2 = _sorted_groups(pp2, wide)
    with _jcfg.enable_x64(False):
        sums2, cnts2, seg2 = _segsum(pp2, ord2, segs2)
        h2 = _mean_matmul(sums2[0], sums2[1], cnts2[0], cnts2[1],
                          W2.T, b2, do_relu=False)
        out = _rowgather(h2, seg2.reshape(NT, PCH, PPC))
    return out[:N]


# speedup vs baseline: 1.3442x; 1.3442x over previous
import functools

import jax
import jax.numpy as jnp
from jax import lax
from jax.experimental import pallas as pl
from jax.experimental.pallas import tpu as pltpu
from jax.experimental.pallas import tpu_sc as plsc
from jax._src import config as _jcfg

N = 10000
D = 128
E = 320000
PARAM_H = 1

NT = 32
NP = 10240
EPC = 128
ECH = 80
EPT = ECH * EPC
PPC = 64
PCH = 5
RPT = NP // 16

_f32 = jnp.float32
_i32 = jnp.int32


def _mesh():
    return plsc.VectorSubcoreMesh(core_axis_name="c", subcore_axis_name="s")


def _i(v):
    v = jnp.asarray(v)
    return v if v.dtype == _i32 else v.astype(_i32)


def _zero_rows(buf, nrows):
    def body(i, _):
        i = _i(i)
        for r in range(D // 16):
            buf[i, pl.ds(r * 16, 16)] = jnp.zeros((16,), _f32)
        return 0
    lax.fori_loop(0, nrows, body, 0)


SB = 16
ECH0 = 128
ECH1 = 32
EPT0 = ECH0 * EPC
EPT1 = ECH1 * EPC
EPAD = 16 * (EPT0 + EPT1)


@functools.partial(
    pl.kernel,
    out_type=jax.ShapeDtypeStruct((2, NP, D), _f32),
    mesh=_mesh(),
    compiler_params=pltpu.CompilerParams(needs_layout_passes=False),
    scratch_types=[
        pltpu.VMEM_SHARED((NP, D), _f32),
        pltpu.VMEM((SB, EPC), _i32),
        pltpu.VMEM((SB, EPC), _i32),
        pltpu.VMEM((SB * EPC,), _f32),
        pltpu.VMEM((EPC, D), _f32),
        pltpu.VMEM((EPC, D), _f32),
        pltpu.SemaphoreType.DMA,
        pltpu.SemaphoreType.DMA,
        pltpu.SemaphoreType.DMA,
        pltpu.SemaphoreType.DMA,
    ],
)
def _propagate(table, srcA, dstA, wA, srcB, dstB, wB, out,
               accum, sidx, didx, wv, rows0, rows1, g0, g1, s0, s1):
    c = lax.axis_index("c")
    s = lax.axis_index("s")

    _zero_rows(rows0, EPC)
    base = s * RPT
    for kk in range(RPT // EPC):
        pltpu.sync_copy(rows0, accum.at[pl.ds(base + kk * EPC, EPC)])
    plsc.subcore_barrier()

    def scale(buf, cb):
        def edge(e):
            wspl = plsc.load_gather(wv, [jnp.full((16,), cb * EPC + e, _i32)])
            for r in range(D // 16):
                buf[e, pl.ds(r * 16, 16)] = buf[e, pl.ds(r * 16, 16)] * wspl
        plsc.parallel_loop(0, EPC, unroll=8)(edge)

    def run_core(src3, dst3, w3, nblk):
        def block(blk, _):
            blk = _i(blk)
            pltpu.sync_copy(src3.at[s, pl.ds(blk * SB, SB)], sidx)
            pltpu.sync_copy(dst3.at[s, pl.ds(blk * SB, SB)], didx)
            pltpu.sync_copy(w3.at[s, pl.ds(blk * (SB * EPC), SB * EPC)], wv)

            bufs = (rows0, rows1)
            gsems = (g0, g1)
            ssems = (s0, s1)
            gd = [pltpu.async_copy(table.at[sidx.at[0]], rows0, g0),
                  pltpu.async_copy(table.at[sidx.at[1]], rows1, g1)]
            for t in range(SB // 2):
                sd = [None, None]
                for b in range(2):
                    cb = 2 * t + b
                    gd[b].wait()
                    scale(bufs[b], cb)
                    sd[b] = pltpu.async_copy(
                        bufs[b], accum.at[didx.at[cb]], ssems[b], add=True)
                for b in range(2):
                    sd[b].wait()
                    if t < SB // 2 - 1:
                        gd[b] = pltpu.async_copy(
                            table.at[sidx.at[2 * t + 2 + b]], bufs[b],
                            gsems[b])
            return 0
        lax.fori_loop(0, nblk, block, 0)

    @pl.when(c == 0)
    def _():
        run_core(srcA, dstA, wA, ECH0 // SB)

    @pl.when(c == 1)
    def _():
        run_core(srcB, dstB, wB, ECH1 // SB)

    plsc.subcore_barrier()
    pltpu.sync_copy(accum.at[pl.ds(base, RPT)], out.at[c, pl.ds(base, RPT)])


@functools.partial(
    pl.kernel,
    out_type=jax.ShapeDtypeStruct((NT, EPT), _i32),
    mesh=_mesh(),
    compiler_params=pltpu.CompilerParams(needs_layout_passes=False),
    scratch_types=[
        pltpu.VMEM((NP,), _i32),
        pltpu.VMEM((EPT,), _i32),
        pltpu.VMEM((EPT,), _i32),
    ],
)
def _compose(seg, src3, out, segtab, sidx, res):
    c = lax.axis_index("c")
    s = lax.axis_index("s")
    wid = c * 16 + s
    pltpu.sync_copy(seg, segtab)
    pltpu.sync_copy(src3.at[wid], sidx)

    def body(t):
        sv = sidx[pl.ds(t * 16, 16)]
        res[pl.ds(t * 16, 16)] = plsc.load_gather(segtab, [sv])
    plsc.parallel_loop(0, EPT // 16, unroll=8)(body)
    pltpu.sync_copy(res, out.at[wid])


@functools.partial(
    pl.kernel,
    out_type=(
        jax.ShapeDtypeStruct((2, NP, D), _f32),
        jax.ShapeDtypeStruct((2, NP), _f32),
        jax.ShapeDtypeStruct((NP,), _i32),
    ),
    mesh=_mesh(),
    compiler_params=pltpu.CompilerParams(needs_layout_passes=False),
    scratch_types=[
        pltpu.VMEM_SHARED((NP, D), _f32),
        pltpu.VMEM_SHARED((NP,), _f32),
        pltpu.VMEM((PCH, PPC), _i32),
        pltpu.VMEM((PCH, PPC), _i32),
        pltpu.VMEM((PPC, D), _f32),
        pltpu.VMEM((PPC, D), _f32),
        pltpu.VMEM((PPC,), _f32),
        pltpu.VMEM((RPT,), _f32),
    ],
)
def _segsum(p0, p1, ord3, seg3, sums, cnts, segarr,
            accum, cacc, ordv, segv, buf, buf2, ones, zbuf):
    c = lax.axis_index("c")
    s = lax.axis_index("s")
    wid = c * 16 + s

    _zero_rows(buf, PPC)
    def zb(i, _):
        i = _i(i)
        zbuf[pl.ds(i * 16, 16)] = jnp.zeros((16,), _f32)
        return 0
    lax.fori_loop(0, RPT // 16, zb, 0)
    for r in range(PPC // 16):
        ones[pl.ds(r * 16, 16)] = jnp.ones((16,), _f32)

    base = s * RPT
    for kk in range(RPT // PPC):
        pltpu.sync_copy(buf, accum.at[pl.ds(base + kk * PPC, PPC)])
    pltpu.sync_copy(zbuf, cacc.at[pl.ds(base, RPT)])

    pltpu.sync_copy(ord3.at[wid], ordv)
    pltpu.sync_copy(seg3.at[wid], segv)
    plsc.subcore_barrier()

    for k in range(PCH):
        pltpu.sync_copy(p0.at[ordv.at[k]], buf)
        pltpu.sync_copy(p1.at[ordv.at[k]], buf2)

        def row(e, _):
            e = _i(e)
            for r in range(D // 16):
                buf[e, pl.ds(r * 16, 16)] = (
                    buf[e, pl.ds(r * 16, 16)] + buf2[e, pl.ds(r * 16, 16)])
            return 0
        lax.fori_loop(0, PPC, row, 0)

        pltpu.sync_copy(buf, accum.at[segv.at[k]], add=True)
        pltpu.sync_copy(ones, cacc.at[segv.at[k]], add=True)
        pltpu.sync_copy(segv.at[k], segarr.at[ordv.at[k]])

    plsc.subcore_barrier()
    pltpu.sync_copy(accum.at[pl.ds(base, RPT)], sums.at[c, pl.ds(base, RPT)])
    pltpu.sync_copy(cacc.at[pl.ds(base, RPT)], cnts.at[c, pl.ds(base, RPT)])


@functools.partial(
    pl.kernel,
    out_type=jax.ShapeDtypeStruct((NP, D), _f32),
    mesh=_mesh(),
    compiler_params=pltpu.CompilerParams(needs_layout_passes=False),
    scratch_types=[
        pltpu.VMEM((PCH, PPC), _i32),
        pltpu.VMEM((PPC, D), _f32),
    ],
)
def _rowgather(table, idx3, out, idxv, buf):
    c = lax.axis_index("c")
    s = lax.axis_index("s")
    wid = c * 16 + s
    pltpu.sync_copy(idx3.at[wid], idxv)
    base = wid * (PCH * PPC)
    for k in range(PCH):
        pltpu.sync_copy(table.at[idxv.at[k]], buf)
        pltpu.sync_copy(buf, out.at[pl.ds(base + k * PPC, PPC)])


_MM_BLOCK = 640


def _mm_kernel(do_relu, s0_ref, s1_ref, c0_ref, c1_ref, wt_ref, b_ref, o_ref):
    cnt = jnp.maximum(c0_ref[...] + c1_ref[...], 1.0)
    xs = (s0_ref[...] + s1_ref[...]) / cnt
    acc = jnp.dot(xs, wt_ref[...], preferred_element_type=jnp.float32)
    acc = acc + b_ref[...][None, :]
    if do_relu:
        acc = jnp.maximum(acc, 0.0)
    o_ref[...] = acc


def _c0():
    return jnp.zeros((), jnp.int32)


def _mean_matmul(s0, s1, c0, c1, Wt, b, do_relu):
    n = s0.shape[0]
    grid = n // _MM_BLOCK
    return pl.pallas_call(
        functools.partial(_mm_kernel, do_relu),
        grid=(grid,),
        in_specs=[
            pl.BlockSpec((_MM_BLOCK, D), lambda i: (i, _c0())),
            pl.BlockSpec((_MM_BLOCK, D), lambda i: (i, _c0())),
            pl.BlockSpec((_MM_BLOCK, 1), lambda i: (i, _c0())),
            pl.BlockSpec((_MM_BLOCK, 1), lambda i: (i, _c0())),
            pl.BlockSpec((D, D), lambda i: (_c0(), _c0())),
            pl.BlockSpec((D,), lambda i: (_c0(),)),
        ],
        out_specs=pl.BlockSpec((_MM_BLOCK, D), lambda i: (i, _c0())),
        out_shape=jax.ShapeDtypeStruct((n, D), jnp.float32),
    )(s0, s1, c0[:, None], c1[:, None], Wt, b)


def _sorted_groups(p0, p1, wide):
    h = p0[:N] + p1[:N]
    itype = jnp.int64 if wide else jnp.int32
    q = jnp.round(h * PARAM_H).astype(itype)
    wts = jnp.arange(D, dtype=itype) * itype(2654435761) + itype(1)
    code = (q * wts).sum(axis=1)
    if wide:
        lo = code.astype(jnp.int32)
        hi = (code >> 32).astype(jnp.int32)
    else:
        lo = code
        hi = jnp.zeros((N,), jnp.int32)
    iota = jnp.arange(N, dtype=jnp.int32)
    lo_s, hi_s, order = lax.sort((lo, hi, iota), num_keys=1)
    newseg = ((lo_s[1:] != lo_s[:-1]) | (hi_s[1:] != hi_s[:-1]))
    seg_sorted = jnp.concatenate(
        [jnp.zeros((1,), jnp.int32), jnp.cumsum(newseg.astype(jnp.int32))])
    pad = jnp.arange(N, NP, dtype=jnp.int32)
    order_p = jnp.concatenate([order, pad]).reshape(NT, PCH, PPC)
    seg_p = jnp.concatenate([seg_sorted, pad]).reshape(NT, PCH, PPC)
    return order_p, seg_p


def kernel(x, edge_index, edge_weight, vertex_cnt, rule_cnt, W1, b1, W2, b2):
    wide = edge_index.dtype == jnp.int64
    with _jcfg.enable_x64(False):
        src = edge_index[0].astype(jnp.int32)
        dst = edge_index[1].astype(jnp.int32)
        epad = EPAD - E
        srcflat = jnp.concatenate([src, jnp.zeros((epad,), jnp.int32)])
        dstflat = jnp.concatenate([dst, jnp.zeros((epad,), jnp.int32)])
        wflat = jnp.concatenate([edge_weight, jnp.zeros((epad,),
                                                        jnp.float32)])
        E0 = 16 * EPT0
        srcf = srcflat.reshape(NT, EPT)
        srcA = srcflat[:E0].reshape(16, ECH0, EPC)
        srcB = srcflat[E0:].reshape(16, ECH1, EPC)
        dstA = dstflat[:E0].reshape(16, ECH0, EPC)
        dstB = dstflat[E0:].reshape(16, ECH1, EPC)
        wA = wflat[:E0].reshape(16, EPT0)
        wB = wflat[E0:].reshape(16, EPT1)

        pp = _propagate(x, srcA, dstA, wA, srcB, dstB, wB)
    ord1, segs1 = _sorted_groups(pp[0], pp[1], wide)
    with _jcfg.enable_x64(False):
        sums1, cnts1, seg1 = _segsum(pp[0], pp[1], ord1, segs1)
        src2flat = _compose(seg1, srcf).reshape(EPAD)
        src2A = src2flat[:E0].reshape(16, ECH0, EPC)
        src2B = src2flat[E0:].reshape(16, ECH1, EPC)
        h1 = _mean_matmul(sums1[0], sums1[1], cnts1[0], cnts1[1],
                          W1.T, b1, do_relu=True)
        pp2 = _propagate(h1, src2A, dstA, wA, src2B, dstB, wB)
    ord2, segs2 = _sorted_groups(pp2[0], pp2[1], wide)
    with _jcfg.enable_x64(False):
        sums2, cnts2, seg2 = _segsum(pp2[0], pp2[1], ord2, segs2)
        h2 = _mean_matmul(sums2[0], sums2[1], cnts2[0], cnts2[1],
                          W2.T, b2, do_relu=False)
        out = _rowgather(h2, seg2.reshape(NT, PCH, PPC))
    return out[:N]


# speedup vs baseline: 1.3500x; 1.0043x over previous
import functools

import jax
import jax.numpy as jnp
from jax import lax
from jax.experimental import pallas as pl
from jax.experimental.pallas import tpu as pltpu
from jax.experimental.pallas import tpu_sc as plsc
from jax._src import config as _jcfg

N = 10000
D = 128
E = 320000
PARAM_H = 1

NT = 32
NP = 10240
EPC = 128
ECH = 80
EPT = ECH * EPC
PPC = 64
PCH = 5
RPT = NP // 16

_f32 = jnp.float32
_i32 = jnp.int32


def _mesh():
    return plsc.VectorSubcoreMesh(core_axis_name="c", subcore_axis_name="s")


def _i(v):
    v = jnp.asarray(v)
    return v if v.dtype == _i32 else v.astype(_i32)


def _zero_rows(buf, nrows):
    def body(i, _):
        i = _i(i)
        for r in range(D // 16):
            buf[i, pl.ds(r * 16, 16)] = jnp.zeros((16,), _f32)
        return 0
    lax.fori_loop(0, nrows, body, 0)


SB = 16
ECH0 = 128
ECH1 = 32
EPT0 = ECH0 * EPC
EPT1 = ECH1 * EPC
EPAD = 16 * (EPT0 + EPT1)


@functools.partial(
    pl.kernel,
    out_type=jax.ShapeDtypeStruct((2, NP, D), _f32),
    mesh=_mesh(),
    compiler_params=pltpu.CompilerParams(needs_layout_passes=False),
    scratch_types=[
        pltpu.VMEM_SHARED((NP, D), _f32),
        pltpu.VMEM((SB, EPC), _i32),
        pltpu.VMEM((SB, EPC), _i32),
        pltpu.VMEM((SB * EPC,), _f32),
        pltpu.VMEM((EPC, D), _f32),
        pltpu.VMEM((EPC, D), _f32),
        pltpu.SemaphoreType.DMA,
        pltpu.SemaphoreType.DMA,
        pltpu.SemaphoreType.DMA,
        pltpu.SemaphoreType.DMA,
    ],
)
def _propagate(table, srcA, dstA, wA, srcB, dstB, wB, out,
               accum, sidx, didx, wv, rows0, rows1, g0, g1, s0, s1):
    c = lax.axis_index("c")
    s = lax.axis_index("s")

    _zero_rows(rows0, EPC)
    base = s * RPT
    for kk in range(RPT // EPC):
        pltpu.sync_copy(rows0, accum.at[pl.ds(base + kk * EPC, EPC)])
    plsc.subcore_barrier()

    def scale(buf, cb):
        def edge(e):
            wspl = plsc.load_gather(wv, [jnp.full((16,), cb * EPC + e, _i32)])
            for r in range(D // 16):
                buf[e, pl.ds(r * 16, 16)] = buf[e, pl.ds(r * 16, 16)] * wspl
        plsc.parallel_loop(0, EPC, unroll=8)(edge)

    def run_core(src3, dst3, w3, nblk):
        def block(blk, _):
            blk = _i(blk)
            pltpu.sync_copy(src3.at[s, pl.ds(blk * SB, SB)], sidx)
            pltpu.sync_copy(dst3.at[s, pl.ds(blk * SB, SB)], didx)
            pltpu.sync_copy(w3.at[s, pl.ds(blk * (SB * EPC), SB * EPC)], wv)

            bufs = (rows0, rows1)
            gsems = (g0, g1)
            ssems = (s0, s1)
            gd = [pltpu.async_copy(table.at[sidx.at[0]], rows0, g0),
                  pltpu.async_copy(table.at[sidx.at[1]], rows1, g1)]
            for t in range(SB // 2):
                sd = [None, None]
                for b in range(2):
                    cb = 2 * t + b
                    gd[b].wait()
                    scale(bufs[b], cb)
                    sd[b] = pltpu.async_copy(
                        bufs[b], accum.at[didx.at[cb]], ssems[b], add=True)
                for b in range(2):
                    sd[b].wait()
                    if t < SB // 2 - 1:
                        gd[b] = pltpu.async_copy(
                            table.at[sidx.at[2 * t + 2 + b]], bufs[b],
                            gsems[b])
            return 0
        lax.fori_loop(0, nblk, block, 0)

    def run_core_dyn(src3, dst3, w3, nblk):
        def block(blk, _):
            blk = _i(blk)
            pltpu.sync_copy(src3.at[s, pl.ds(blk * SB, SB)], sidx)
            pltpu.sync_copy(dst3.at[s, pl.ds(blk * SB, SB)], didx)
            pltpu.sync_copy(w3.at[s, pl.ds(blk * (SB * EPC), SB * EPC)], wv)

            bufs = (rows0, rows1)
            gsems = (g0, g1)
            ssems = (s0, s1)
            pltpu.async_copy(table.at[sidx.at[0]], rows0, g0)
            pltpu.async_copy(table.at[sidx.at[1]], rows1, g1)

            def pair(t, _):
                t = _i(t)
                sd = [None, None]
                for b in range(2):
                    cb = 2 * t + b
                    pltpu.make_async_copy(
                        table.at[sidx.at[cb]], bufs[b], gsems[b]).wait()
                    scale(bufs[b], cb)
                    sd[b] = pltpu.async_copy(
                        bufs[b], accum.at[didx.at[cb]], ssems[b], add=True)
                for b in range(2):
                    sd[b].wait()

                    @pl.when(t < SB // 2 - 1)
                    def _():
                        pltpu.async_copy(
                            table.at[sidx.at[2 * t + 2 + b]], bufs[b],
                            gsems[b])
                return 0
            lax.fori_loop(0, SB // 2, pair, 0)
            return 0
        lax.fori_loop(0, nblk, block, 0)

    @pl.when(c == 0)
    def _():
        run_core(srcA, dstA, wA, ECH0 // SB)

    @pl.when(c == 1)
    def _():
        run_core_dyn(srcB, dstB, wB, ECH1 // SB)

    plsc.subcore_barrier()
    pltpu.sync_copy(accum.at[pl.ds(base, RPT)], out.at[c, pl.ds(base, RPT)])


@functools.partial(
    pl.kernel,
    out_type=jax.ShapeDtypeStruct((NT, EPT), _i32),
    mesh=_mesh(),
    compiler_params=pltpu.CompilerParams(needs_layout_passes=False),
    scratch_types=[
        pltpu.VMEM((NP,), _i32),
        pltpu.VMEM((EPT,), _i32),
        pltpu.VMEM((EPT,), _i32),
    ],
)
def _compose(seg, src3, out, segtab, sidx, res):
    c = lax.axis_index("c")
    s = lax.axis_index("s")
    wid = c * 16 + s
    pltpu.sync_copy(seg, segtab)
    pltpu.sync_copy(src3.at[wid], sidx)

    def body(t):
        sv = sidx[pl.ds(t * 16, 16)]
        res[pl.ds(t * 16, 16)] = plsc.load_gather(segtab, [sv])
    plsc.parallel_loop(0, EPT // 16, unroll=8)(body)
    pltpu.sync_copy(res, out.at[wid])


@functools.partial(
    pl.kernel,
    out_type=(
        jax.ShapeDtypeStruct((2, NP, D), _f32),
        jax.ShapeDtypeStruct((2, NP), _f32),
        jax.ShapeDtypeStruct((NP,), _i32),
    ),
    mesh=_mesh(),
    compiler_params=pltpu.CompilerParams(needs_layout_passes=False),
    scratch_types=[
        pltpu.VMEM_SHARED((NP, D), _f32),
        pltpu.VMEM_SHARED((NP,), _f32),
        pltpu.VMEM((PCH, PPC), _i32),
        pltpu.VMEM((PCH, PPC), _i32),
        pltpu.VMEM((PPC, D), _f32),
        pltpu.VMEM((PPC, D), _f32),
        pltpu.VMEM((PPC,), _f32),
        pltpu.VMEM((RPT,), _f32),
    ],
)
def _segsum(p0, p1, ord3, seg3, sums, cnts, segarr,
            accum, cacc, ordv, segv, buf, buf2, ones, zbuf):
    c = lax.axis_index("c")
    s = lax.axis_index("s")
    wid = c * 16 + s

    _zero_rows(buf, PPC)
    def zb(i, _):
        i = _i(i)
        zbuf[pl.ds(i * 16, 16)] = jnp.zeros((16,), _f32)
        return 0
    lax.fori_loop(0, RPT // 16, zb, 0)
    for r in range(PPC // 16):
        ones[pl.ds(r * 16, 16)] = jnp.ones((16,), _f32)

    base = s * RPT
    for kk in range(RPT // PPC):
        pltpu.sync_copy(buf, accum.at[pl.ds(base + kk * PPC, PPC)])
    pltpu.sync_copy(zbuf, cacc.at[pl.ds(base, RPT)])

    pltpu.sync_copy(ord3.at[wid], ordv)
    pltpu.sync_copy(seg3.at[wid], segv)
    plsc.subcore_barrier()

    for k in range(PCH):
        pltpu.sync_copy(p0.at[ordv.at[k]], buf)
        pltpu.sync_copy(p1.at[ordv.at[k]], buf2)

        def row(e, _):
            e = _i(e)
            for r in range(D // 16):
                buf[e, pl.ds(r * 16, 16)] = (
                    buf[e, pl.ds(r * 16, 16)] + buf2[e, pl.ds(r * 16, 16)])
            return 0
        lax.fori_loop(0, PPC, row, 0)

        pltpu.sync_copy(buf, accum.at[segv.at[k]], add=True)
        pltpu.sync_copy(ones, cacc.at[segv.at[k]], add=True)
        pltpu.sync_copy(segv.at[k], segarr.at[ordv.at[k]])

    plsc.subcore_barrier()
    pltpu.sync_copy(accum.at[pl.ds(base, RPT)], sums.at[c, pl.ds(base, RPT)])
    pltpu.sync_copy(cacc.at[pl.ds(base, RPT)], cnts.at[c, pl.ds(base, RPT)])


@functools.partial(
    pl.kernel,
    out_type=jax.ShapeDtypeStruct((NP, D), _f32),
    mesh=_mesh(),
    compiler_params=pltpu.CompilerParams(needs_layout_passes=False),
    scratch_types=[
        pltpu.VMEM((PCH, PPC), _i32),
        pltpu.VMEM((PPC, D), _f32),
    ],
)
def _rowgather(table, idx3, out, idxv, buf):
    c = lax.axis_index("c")
    s = lax.axis_index("s")
    wid = c * 16 + s
    pltpu.sync_copy(idx3.at[wid], idxv)
    base = wid * (PCH * PPC)
    for k in range(PCH):
        pltpu.sync_copy(table.at[idxv.at[k]], buf)
        pltpu.sync_copy(buf, out.at[pl.ds(base + k * PPC, PPC)])


_MM_BLOCK = 640


def _mm_kernel(do_relu, s0_ref, s1_ref, c0_ref, c1_ref, wt_ref, b_ref, o_ref):
    cnt = jnp.maximum(c0_ref[...] + c1_ref[...], 1.0)
    xs = (s0_ref[...] + s1_ref[...]) / cnt
    acc = jnp.dot(xs, wt_ref[...], preferred_element_type=jnp.float32)
    acc = acc + b_ref[...][None, :]
    if do_relu:
        acc = jnp.maximum(acc, 0.0)
    o_ref[...] = acc


def _c0():
    return jnp.zeros((), jnp.int32)


def _mean_matmul(s0, s1, c0, c1, Wt, b, do_relu):
    n = s0.shape[0]
    grid = n // _MM_BLOCK
    return pl.pallas_call(
        functools.partial(_mm_kernel, do_relu),
        grid=(grid,),
        in_specs=[
            pl.BlockSpec((_MM_BLOCK, D), lambda i: (i, _c0())),
            pl.BlockSpec((_MM_BLOCK, D), lambda i: (i, _c0())),
            pl.BlockSpec((_MM_BLOCK, 1), lambda i: (i, _c0())),
            pl.BlockSpec((_MM_BLOCK, 1), lambda i: (i, _c0())),
            pl.BlockSpec((D, D), lambda i: (_c0(), _c0())),
            pl.BlockSpec((D,), lambda i: (_c0(),)),
        ],
        out_specs=pl.BlockSpec((_MM_BLOCK, D), lambda i: (i, _c0())),
        out_shape=jax.ShapeDtypeStruct((n, D), jnp.float32),
    )(s0, s1, c0[:, None], c1[:, None], Wt, b)


def _sorted_groups(p0, p1, wide):
    h = p0[:N] + p1[:N]
    itype = jnp.int64 if wide else jnp.int32
    q = jnp.round(h * PARAM_H).astype(itype)
    wts = jnp.arange(D, dtype=itype) * itype(2654435761) + itype(1)
    code = (q * wts).sum(axis=1)
    if wide:
        lo = code.astype(jnp.int32)
        hi = (code >> 32).astype(jnp.int32)
    else:
        lo = code
        hi = jnp.zeros((N,), jnp.int32)
    iota = jnp.arange(N, dtype=jnp.int32)
    lo_s, hi_s, order = lax.sort((lo, hi, iota), num_keys=1)
    newseg = ((lo_s[1:] != lo_s[:-1]) | (hi_s[1:] != hi_s[:-1]))
    seg_sorted = jnp.concatenate(
        [jnp.zeros((1,), jnp.int32), jnp.cumsum(newseg.astype(jnp.int32))])
    pad = jnp.arange(N, NP, dtype=jnp.int32)
    order_p = jnp.concatenate([order, pad]).reshape(NT, PCH, PPC)
    seg_p = jnp.concatenate([seg_sorted, pad]).reshape(NT, PCH, PPC)
    return order_p, seg_p


def kernel(x, edge_index, edge_weight, vertex_cnt, rule_cnt, W1, b1, W2, b2):
    wide = edge_index.dtype == jnp.int64
    with _jcfg.enable_x64(False):
        src = edge_index[0].astype(jnp.int32)
        dst = edge_index[1].astype(jnp.int32)
        epad = EPAD - E
        srcflat = jnp.concatenate([src, jnp.zeros((epad,), jnp.int32)])
        dstflat = jnp.concatenate([dst, jnp.zeros((epad,), jnp.int32)])
        wflat = jnp.concatenate([edge_weight, jnp.zeros((epad,),
                                                        jnp.float32)])
        E0 = 16 * EPT0
        srcf = srcflat.reshape(NT, EPT)
        srcA = srcflat[:E0].reshape(16, ECH0, EPC)
        srcB = srcflat[E0:].reshape(16, ECH1, EPC)
        dstA = dstflat[:E0].reshape(16, ECH0, EPC)
        dstB = dstflat[E0:].reshape(16, ECH1, EPC)
        wA = wflat[:E0].reshape(16, EPT0)
        wB = wflat[E0:].reshape(16, EPT1)

        pp = _propagate(x, srcA, dstA, wA, srcB, dstB, wB)
    ord1, segs1 = _sorted_groups(pp[0], pp[1], wide)
    with _jcfg.enable_x64(False):
        sums1, cnts1, seg1 = _segsum(pp[0], pp[1], ord1, segs1)
        src2flat = _compose(seg1, srcf).reshape(EPAD)
        src2A = src2flat[:E0].reshape(16, ECH0, EPC)
        src2B = src2flat[E0:].reshape(16, ECH1, EPC)
        h1 = _mean_matmul(sums1[0], sums1[1], cnts1[0], cnts1[1],
                          W1.T, b1, do_relu=True)
        pp2 = _propagate(h1, src2A, dstA, wA, src2B, dstB, wB)
    ord2, segs2 = _sorted_groups(pp2[0], pp2[1], wide)
    with _jcfg.enable_x64(False):
        sums2, cnts2, seg2 = _segsum(pp2[0], pp2[1], ord2, segs2)
        h2 = _mean_matmul(sums2[0], sums2[1], cnts2[0], cnts2[1],
                          W2.T, b2, do_relu=False)
        out = _rowgather(h2, seg2.reshape(NT, PCH, PPC))
    return out[:N]


# speedup vs baseline: 1.5393x; 1.1402x over previous
import functools

import jax
import jax.numpy as jnp
from jax import lax
from jax.experimental import pallas as pl
from jax.experimental.pallas import tpu as pltpu
from jax.experimental.pallas import tpu_sc as plsc
from jax._src import config as _jcfg

N = 10000
D = 128
E = 320000
PARAM_H = 1

NT = 32
NP = 10240
EPC = 128
ECH = 80
EPT = ECH * EPC
PPC = 64
PCH = 5
RPT = NP // 16

_f32 = jnp.float32
_i32 = jnp.int32


def _mesh():
    return plsc.VectorSubcoreMesh(core_axis_name="c", subcore_axis_name="s")


def _i(v):
    v = jnp.asarray(v)
    return v if v.dtype == _i32 else v.astype(_i32)


def _zero_rows(buf, nrows):
    def body(i, _):
        i = _i(i)
        for r in range(D // 16):
            buf[i, pl.ds(r * 16, 16)] = jnp.zeros((16,), _f32)
        return 0
    lax.fori_loop(0, nrows, body, 0)


SB = 16
ECH0 = 144
ECH1 = 16
EPT0 = ECH0 * EPC
EPT1 = ECH1 * EPC
EPAD = 16 * (EPT0 + EPT1)


@functools.partial(
    pl.kernel,
    out_type=jax.ShapeDtypeStruct((2, NP, D), _f32),
    mesh=_mesh(),
    compiler_params=pltpu.CompilerParams(needs_layout_passes=False),
    scratch_types=[
        pltpu.VMEM_SHARED((NP, D), _f32),
        pltpu.VMEM((SB, EPC), _i32),
        pltpu.VMEM((SB, EPC), _i32),
        pltpu.VMEM((SB * EPC,), _f32),
        pltpu.VMEM((EPC, D), _f32),
        pltpu.VMEM((EPC, D), _f32),
        pltpu.SemaphoreType.DMA,
        pltpu.SemaphoreType.DMA,
        pltpu.SemaphoreType.DMA,
        pltpu.SemaphoreType.DMA,
    ],
)
def _propagate(table, srcA, dstA, wA, srcB, dstB, wB, out,
               accum, sidx, didx, wv, rows0, rows1, g0, g1, s0, s1):
    c = lax.axis_index("c")
    s = lax.axis_index("s")

    _zero_rows(rows0, EPC)
    base = s * RPT
    for kk in range(RPT // EPC):
        pltpu.sync_copy(rows0, accum.at[pl.ds(base + kk * EPC, EPC)])
    plsc.subcore_barrier()

    def scale(buf, cb):
        def edge(e):
            wspl = plsc.load_gather(wv, [jnp.full((16,), cb * EPC + e, _i32)])
            for r in range(D // 16):
                buf[e, pl.ds(r * 16, 16)] = buf[e, pl.ds(r * 16, 16)] * wspl
        plsc.parallel_loop(0, EPC, unroll=8)(edge)

    def run_core(src3, dst3, w3, nblk):
        def block(blk, _):
            blk = _i(blk)
            pltpu.sync_copy(src3.at[s, pl.ds(blk * SB, SB)], sidx)
            pltpu.sync_copy(dst3.at[s, pl.ds(blk * SB, SB)], didx)
            pltpu.sync_copy(w3.at[s, pl.ds(blk * (SB * EPC), SB * EPC)], wv)

            bufs = (rows0, rows1)
            gsems = (g0, g1)
            ssems = (s0, s1)
            gd = [pltpu.async_copy(table.at[sidx.at[0]], rows0, g0),
                  pltpu.async_copy(table.at[sidx.at[1]], rows1, g1)]
            for t in range(SB // 2):
                sd = [None, None]
                for b in range(2):
                    cb = 2 * t + b
                    gd[b].wait()
                    scale(bufs[b], cb)
                    sd[b] = pltpu.async_copy(
                        bufs[b], accum.at[didx.at[cb]], ssems[b], add=True)
                for b in range(2):
                    sd[b].wait()
                    if t < SB // 2 - 1:
                        gd[b] = pltpu.async_copy(
                            table.at[sidx.at[2 * t + 2 + b]], bufs[b],
                            gsems[b])
            return 0
        lax.fori_loop(0, nblk, block, 0)

    def run_core_dyn(src3, dst3, w3, nblk):
        def block(blk, _):
            blk = _i(blk)
            pltpu.sync_copy(src3.at[s, pl.ds(blk * SB, SB)], sidx)
            pltpu.sync_copy(dst3.at[s, pl.ds(blk * SB, SB)], didx)
            pltpu.sync_copy(w3.at[s, pl.ds(blk * (SB * EPC), SB * EPC)], wv)

            bufs = (rows0, rows1)
            gsems = (g0, g1)
            ssems = (s0, s1)
            pltpu.async_copy(table.at[sidx.at[0]], rows0, g0)
            pltpu.async_copy(table.at[sidx.at[1]], rows1, g1)

            def pair(t, _):
                t = _i(t)
                sd = [None, None]
                for b in range(2):
                    cb = 2 * t + b
                    pltpu.make_async_copy(
                        table.at[sidx.at[cb]], bufs[b], gsems[b]).wait()
                    scale(bufs[b], cb)
                    sd[b] = pltpu.async_copy(
                        bufs[b], accum.at[didx.at[cb]], ssems[b], add=True)
                for b in range(2):
                    sd[b].wait()

                    @pl.when(t < SB // 2 - 1)
                    def _():
                        pltpu.async_copy(
                            table.at[sidx.at[2 * t + 2 + b]], bufs[b],
                            gsems[b])
                return 0
            lax.fori_loop(0, SB // 2, pair, 0)
            return 0
        lax.fori_loop(0, nblk, block, 0)

    @pl.when(c == 0)
    def _():
        run_core(srcA, dstA, wA, ECH0 // SB)

    @pl.when(c == 1)
    def _():
        run_core_dyn(srcB, dstB, wB, ECH1 // SB)

    plsc.subcore_barrier()
    pltpu.sync_copy(accum.at[pl.ds(base, RPT)], out.at[c, pl.ds(base, RPT)])


@functools.partial(
    pl.kernel,
    out_type=jax.ShapeDtypeStruct((NT, EPT), _i32),
    mesh=_mesh(),
    compiler_params=pltpu.CompilerParams(needs_layout_passes=False),
    scratch_types=[
        pltpu.VMEM((NP,), _i32),
        pltpu.VMEM((EPT,), _i32),
        pltpu.VMEM((EPT,), _i32),
    ],
)
def _compose(seg, src3, out, segtab, sidx, res):
    c = lax.axis_index("c")
    s = lax.axis_index("s")
    wid = c * 16 + s
    pltpu.sync_copy(seg, segtab)
    pltpu.sync_copy(src3.at[wid], sidx)

    def body(t):
        sv = sidx[pl.ds(t * 16, 16)]
        res[pl.ds(t * 16, 16)] = plsc.load_gather(segtab, [sv])
    plsc.parallel_loop(0, EPT // 16, unroll=8)(body)
    pltpu.sync_copy(res, out.at[wid])


@functools.partial(
    pl.kernel,
    out_type=(
        jax.ShapeDtypeStruct((2, NP, D), _f32),
        jax.ShapeDtypeStruct((2, NP), _f32),
        jax.ShapeDtypeStruct((NP,), _i32),
    ),
    mesh=_mesh(),
    compiler_params=pltpu.CompilerParams(needs_layout_passes=False),
    scratch_types=[
        pltpu.VMEM_SHARED((NP, D), _f32),
        pltpu.VMEM_SHARED((NP,), _f32),
        pltpu.VMEM((PCH, PPC), _i32),
        pltpu.VMEM((PCH, PPC), _i32),
        pltpu.VMEM((PPC, D), _f32),
        pltpu.VMEM((PPC, D), _f32),
        pltpu.VMEM((PPC,), _f32),
        pltpu.VMEM((RPT,), _f32),
    ],
)
def _segsum(p0, p1, ord3, seg3, sums, cnts, segarr,
            accum, cacc, ordv, segv, buf, buf2, ones, zbuf):
    c = lax.axis_index("c")
    s = lax.axis_index("s")
    wid = c * 16 + s

    _zero_rows(buf, PPC)
    def zb(i, _):
        i = _i(i)
        zbuf[pl.ds(i * 16, 16)] = jnp.zeros((16,), _f32)
        return 0
    lax.fori_loop(0, RPT // 16, zb, 0)
    for r in range(PPC // 16):
        ones[pl.ds(r * 16, 16)] = jnp.ones((16,), _f32)

    base = s * RPT
    for kk in range(RPT // PPC):
        pltpu.sync_copy(buf, accum.at[pl.ds(base + kk * PPC, PPC)])
    pltpu.sync_copy(zbuf, cacc.at[pl.ds(base, RPT)])

    pltpu.sync_copy(ord3.at[wid], ordv)
    pltpu.sync_copy(seg3.at[wid], segv)
    plsc.subcore_barrier()

    for k in range(PCH):
        pltpu.sync_copy(p0.at[ordv.at[k]], buf)
        pltpu.sync_copy(p1.at[ordv.at[k]], buf2)

        def row(e, _):
            e = _i(e)
            for r in range(D // 16):
                buf[e, pl.ds(r * 16, 16)] = (
                    buf[e, pl.ds(r * 16, 16)] + buf2[e, pl.ds(r * 16, 16)])
            return 0
        lax.fori_loop(0, PPC, row, 0)

        pltpu.sync_copy(buf, accum.at[segv.at[k]], add=True)
        pltpu.sync_copy(ones, cacc.at[segv.at[k]], add=True)
        pltpu.sync_copy(segv.at[k], segarr.at[ordv.at[k]])

    plsc.subcore_barrier()
    pltpu.sync_copy(accum.at[pl.ds(base, RPT)], sums.at[c, pl.ds(base, RPT)])
    pltpu.sync_copy(cacc.at[pl.ds(base, RPT)], cnts.at[c, pl.ds(base, RPT)])


@functools.partial(
    pl.kernel,
    out_type=jax.ShapeDtypeStruct((NP, D), _f32),
    mesh=_mesh(),
    compiler_params=pltpu.CompilerParams(needs_layout_passes=False),
    scratch_types=[
        pltpu.VMEM((PCH, PPC), _i32),
        pltpu.VMEM((PPC, D), _f32),
    ],
)
def _rowgather(table, idx3, out, idxv, buf):
    c = lax.axis_index("c")
    s = lax.axis_index("s")
    wid = c * 16 + s
    pltpu.sync_copy(idx3.at[wid], idxv)
    base = wid * (PCH * PPC)
    for k in range(PCH):
        pltpu.sync_copy(table.at[idxv.at[k]], buf)
        pltpu.sync_copy(buf, out.at[pl.ds(base + k * PPC, PPC)])


_MM_BLOCK = 640


def _mm_kernel(do_relu, s0_ref, s1_ref, c0_ref, c1_ref, wt_ref, b_ref, o_ref):
    cnt = jnp.maximum(c0_ref[...] + c1_ref[...], 1.0)
    xs = (s0_ref[...] + s1_ref[...]) / cnt
    acc = jnp.dot(xs, wt_ref[...], preferred_element_type=jnp.float32)
    acc = acc + b_ref[...][None, :]
    if do_relu:
        acc = jnp.maximum(acc, 0.0)
    o_ref[...] = acc


def _c0():
    return jnp.zeros((), jnp.int32)


def _mean_matmul(s0, s1, c0, c1, Wt, b, do_relu):
    n = s0.shape[0]
    grid = n // _MM_BLOCK
    return pl.pallas_call(
        functools.partial(_mm_kernel, do_relu),
        grid=(grid,),
        in_specs=[
            pl.BlockSpec((_MM_BLOCK, D), lambda i: (i, _c0())),
            pl.BlockSpec((_MM_BLOCK, D), lambda i: (i, _c0())),
            pl.BlockSpec((_MM_BLOCK, 1), lambda i: (i, _c0())),
            pl.BlockSpec((_MM_BLOCK, 1), lambda i: (i, _c0())),
            pl.BlockSpec((D, D), lambda i: (_c0(), _c0())),
            pl.BlockSpec((D,), lambda i: (_c0(),)),
        ],
        out_specs=pl.BlockSpec((_MM_BLOCK, D), lambda i: (i, _c0())),
        out_shape=jax.ShapeDtypeStruct((n, D), jnp.float32),
    )(s0, s1, c0[:, None], c1[:, None], Wt, b)


def _sorted_groups(p0, p1, wide):
    h = p0[:N] + p1[:N]
    itype = jnp.int64 if wide else jnp.int32
    q = jnp.round(h * PARAM_H).astype(itype)
    wts = jnp.arange(D, dtype=itype) * itype(2654435761) + itype(1)
    code = (q * wts).sum(axis=1)
    if wide:
        lo = code.astype(jnp.int32)
        hi = (code >> 32).astype(jnp.int32)
    else:
        lo = code
        hi = jnp.zeros((N,), jnp.int32)
    iota = jnp.arange(N, dtype=jnp.int32)
    lo_s, hi_s, order = lax.sort((lo, hi, iota), num_keys=1)
    newseg = ((lo_s[1:] != lo_s[:-1]) | (hi_s[1:] != hi_s[:-1]))
    seg_sorted = jnp.concatenate(
        [jnp.zeros((1,), jnp.int32), jnp.cumsum(newseg.astype(jnp.int32))])
    pad = jnp.arange(N, NP, dtype=jnp.int32)
    order_p = jnp.concatenate([order, pad]).reshape(NT, PCH, PPC)
    seg_p = jnp.concatenate([seg_sorted, pad]).reshape(NT, PCH, PPC)
    return order_p, seg_p


def kernel(x, edge_index, edge_weight, vertex_cnt, rule_cnt, W1, b1, W2, b2):
    wide = edge_index.dtype == jnp.int64
    with _jcfg.enable_x64(False):
        src = edge_index[0].astype(jnp.int32)
        dst = edge_index[1].astype(jnp.int32)
        epad = EPAD - E
        srcflat = jnp.concatenate([src, jnp.zeros((epad,), jnp.int32)])
        dstflat = jnp.concatenate([dst, jnp.zeros((epad,), jnp.int32)])
        wflat = jnp.concatenate([edge_weight, jnp.zeros((epad,),
                                                        jnp.float32)])
        E0 = 16 * EPT0
        srcf = srcflat.reshape(NT, EPT)
        srcA = srcflat[:E0].reshape(16, ECH0, EPC)
        srcB = srcflat[E0:].reshape(16, ECH1, EPC)
        dstA = dstflat[:E0].reshape(16, ECH0, EPC)
        dstB = dstflat[E0:].reshape(16, ECH1, EPC)
        wA = wflat[:E0].reshape(16, EPT0)
        wB = wflat[E0:].reshape(16, EPT1)

        pp = _propagate(x, srcA, dstA, wA, srcB, dstB, wB)
    ord1, segs1 = _sorted_groups(pp[0], pp[1], wide)
    with _jcfg.enable_x64(False):
        sums1, cnts1, seg1 = _segsum(pp[0], pp[1], ord1, segs1)
        src2flat = _compose(seg1, srcf).reshape(EPAD)
        src2A = src2flat[:E0].reshape(16, ECH0, EPC)
        src2B = src2flat[E0:].reshape(16, ECH1, EPC)
        h1 = _mean_matmul(sums1[0], sums1[1], cnts1[0], cnts1[1],
                          W1.T, b1, do_relu=True)
        pp2 = _propagate(h1, src2A, dstA, wA, src2B, dstB, wB)
    ord2, segs2 = _sorted_groups(pp2[0], pp2[1], wide)
    with _jcfg.enable_x64(False):
        sums2, cnts2, seg2 = _segsum(pp2[0], pp2[1], ord2, segs2)
        h2 = _mean_matmul(sums2[0], sums2[1], cnts2[0], cnts2[1],
                          W2.T, b2, do_relu=False)
        out = _rowgather(h2, seg2.reshape(NT, PCH, PPC))
    return out[:N]
